# baseline (device time: 2934928 ns/iter reference)
import jax
import jax.numpy as jnp
from jax import lax
from jax.experimental import pallas as pl
from jax.experimental.pallas import tpu as pltpu

N_DEV = 32


def kernel(x, w_mat):
    m_per, k = x.shape
    _, n_per = w_mat.shape

    def body(x_ref, w_ref, out_ref, comm_ref, send_sems, recv_sems,
             credit_sem):
        my = lax.axis_index("i")
        left = lax.rem(my + N_DEV - 1, N_DEV)
        right = lax.rem(my + 1, N_DEV)

        barrier_sem = pltpu.get_barrier_semaphore()
        for nbr in (left, right):
            pl.semaphore_signal(
                barrier_sem, inc=1,
                device_id=(nbr,), device_id_type=pl.DeviceIdType.MESH,
            )
        pl.semaphore_wait(barrier_sem, 2)

        comm_ref[0] = x_ref[...]
        out_ref[pl.ds(my * m_per, m_per), :] = jnp.dot(
            x_ref[...], w_ref[...], preferred_element_type=jnp.float32)

        for h in range(N_DEV - 1):
            send_slot = h % 2
            recv_slot = (h + 1) % 2
            if h >= 2:
                pl.semaphore_wait(credit_sem, 1)
            rdma = pltpu.make_async_remote_copy(
                src_ref=comm_ref.at[send_slot],
                dst_ref=comm_ref.at[recv_slot],
                send_sem=send_sems.at[send_slot],
                recv_sem=recv_sems.at[recv_slot],
                device_id=(right,),
                device_id_type=pl.DeviceIdType.MESH,
            )
            rdma.start()
            rdma.wait()

            origin = lax.rem(my + 2 * N_DEV - h - 1, N_DEV)
            out_ref[pl.ds(origin * m_per, m_per), :] = jnp.dot(
                comm_ref[recv_slot], w_ref[...],
                preferred_element_type=jnp.float32)
            if h < N_DEV - 3:
                pl.semaphore_signal(
                    credit_sem, inc=1,
                    device_id=(left,), device_id_type=pl.DeviceIdType.MESH,
                )

    out_shape = jax.ShapeDtypeStruct((N_DEV * m_per, n_per), jnp.float32)
    return pl.pallas_call(
        body,
        out_shape=out_shape,
        in_specs=[
            pl.BlockSpec(memory_space=pltpu.VMEM),
            pl.BlockSpec(memory_space=pltpu.VMEM),
        ],
        out_specs=pl.BlockSpec(memory_space=pltpu.VMEM),
        scratch_shapes=[
            pltpu.VMEM((2, m_per, k), jnp.float32),
            pltpu.SemaphoreType.DMA((2,)),
            pltpu.SemaphoreType.DMA((2,)),
            pltpu.SemaphoreType.REGULAR,
        ],
        compiler_params=pltpu.CompilerParams(collective_id=0),
    )(x, w_mat)


# device time: 2926872 ns/iter; 1.0028x vs baseline; 1.0028x over previous
import jax
import jax.numpy as jnp
from jax import lax
from jax.experimental import pallas as pl
from jax.experimental.pallas import tpu as pltpu

N_DEV = 32
R_HOPS = N_DEV // 2
L_HOPS = N_DEV - 1 - R_HOPS


def kernel(x, w_mat):
    m_per, k = x.shape
    _, n_per = w_mat.shape

    def body(x_ref, w_ref, out_ref, rcomm, lcomm,
             r_send_sems, r_recv_sems, l_send_sems, l_recv_sems,
             r_credit, l_credit):
        my = lax.axis_index("i")
        left = lax.rem(my + N_DEV - 1, N_DEV)
        right = lax.rem(my + 1, N_DEV)

        barrier_sem = pltpu.get_barrier_semaphore()
        for nbr in (left, right):
            pl.semaphore_signal(
                barrier_sem, inc=1,
                device_id=(nbr,), device_id_type=pl.DeviceIdType.MESH,
            )
        pl.semaphore_wait(barrier_sem, 2)

        def gemm_to(origin, chunk):
            out_ref[pl.ds(origin * m_per, m_per), :] = jnp.dot(
                chunk, w_ref[...], preferred_element_type=jnp.float32)

        for h in range(R_HOPS):
            s = h % 2
            r = (h + 1) % 2
            l_active = h < L_HOPS
            if h >= 1:
                pl.semaphore_wait(r_credit, 1)
                if l_active:
                    pl.semaphore_wait(l_credit, 1)
            r_src = x_ref if h == 0 else rcomm.at[s]
            r_rdma = pltpu.make_async_remote_copy(
                src_ref=r_src,
                dst_ref=rcomm.at[r],
                send_sem=r_send_sems.at[s],
                recv_sem=r_recv_sems.at[r],
                device_id=(right,),
                device_id_type=pl.DeviceIdType.MESH,
            )
            r_rdma.start()
            if l_active:
                l_src = x_ref if h == 0 else lcomm.at[s]
                l_rdma = pltpu.make_async_remote_copy(
                    src_ref=l_src,
                    dst_ref=lcomm.at[r],
                    send_sem=l_send_sems.at[s],
                    recv_sem=l_recv_sems.at[r],
                    device_id=(left,),
                    device_id_type=pl.DeviceIdType.MESH,
                )
                l_rdma.start()

            if h == 0:
                gemm_to(my, x_ref[...])
            else:
                gemm_to(lax.rem(my + 2 * N_DEV - h, N_DEV), rcomm[s])
                gemm_to(lax.rem(my + h, N_DEV), lcomm[s])

            r_rdma.wait()
            if l_active:
                l_rdma.wait()

            if h < R_HOPS - 1:
                pl.semaphore_signal(
                    r_credit, inc=1,
                    device_id=(left,), device_id_type=pl.DeviceIdType.MESH,
                )
            if l_active and h < L_HOPS - 1:
                pl.semaphore_signal(
                    l_credit, inc=1,
                    device_id=(right,), device_id_type=pl.DeviceIdType.MESH,
                )

        gemm_to(lax.rem(my + 2 * N_DEV - R_HOPS, N_DEV), rcomm[R_HOPS % 2])

    out_shape = jax.ShapeDtypeStruct((N_DEV * m_per, n_per), jnp.float32)
    return pl.pallas_call(
        body,
        out_shape=out_shape,
        in_specs=[
            pl.BlockSpec(memory_space=pltpu.VMEM),
            pl.BlockSpec(memory_space=pltpu.VMEM),
        ],
        out_specs=pl.BlockSpec(memory_space=pltpu.VMEM),
        scratch_shapes=[
            pltpu.VMEM((2, m_per, k), jnp.float32),
            pltpu.VMEM((2, m_per, k), jnp.float32),
            pltpu.SemaphoreType.DMA((2,)),
            pltpu.SemaphoreType.DMA((2,)),
            pltpu.SemaphoreType.DMA((2,)),
            pltpu.SemaphoreType.DMA((2,)),
            pltpu.SemaphoreType.REGULAR,
            pltpu.SemaphoreType.REGULAR,
        ],
        compiler_params=pltpu.CompilerParams(collective_id=0),
    )(x, w_mat)


# device time: 1498307 ns/iter; 1.9588x vs baseline; 1.9535x over previous
import jax
import jax.numpy as jnp
from jax import lax
from jax.experimental import pallas as pl
from jax.experimental.pallas import tpu as pltpu

N_DEV = 32
R_HOPS = N_DEV // 2
L_HOPS = N_DEV - 1 - R_HOPS

PERM = [0, 8, 16, 24, 27, 19, 11, 3, 4, 12, 20, 28, 31, 23, 15, 7,
        6, 14, 22, 30, 29, 21, 13, 5, 2, 10, 18, 26, 25, 17, 9, 1]
INV = [0] * N_DEV
for _pos, _lid in enumerate(PERM):
    INV[_lid] = _pos


def kernel(x, w_mat):
    m_per, k = x.shape
    _, n_per = w_mat.shape

    perm = jnp.asarray(PERM, dtype=jnp.int32)
    inv = jnp.asarray(INV, dtype=jnp.int32)
    my = lax.axis_index("i")
    p = inv[my]
    left = perm[(p - 1) % N_DEV]
    right = perm[(p + 1) % N_DEV]
    r_orig = perm[(p - 1 - jnp.arange(R_HOPS, dtype=jnp.int32)) % N_DEV]
    l_orig = perm[(p + 1 + jnp.arange(L_HOPS, dtype=jnp.int32)) % N_DEV]
    params = jnp.concatenate(
        [jnp.stack([left, right]).astype(jnp.int32), r_orig, l_orig])

    def body(x_ref, w_ref, prm, out_ref, rcomm, lcomm,
             r_send_sems, r_recv_sems, l_send_sems, l_recv_sems,
             r_credit, l_credit):
        my_id = lax.axis_index("i")
        lft = prm[0]
        rgt = prm[1]

        barrier_sem = pltpu.get_barrier_semaphore()
        for nbr in (lft, rgt):
            pl.semaphore_signal(
                barrier_sem, inc=1,
                device_id=(nbr,), device_id_type=pl.DeviceIdType.MESH,
            )
        pl.semaphore_wait(barrier_sem, 2)

        def gemm_to(origin, chunk):
            out_ref[pl.ds(origin * m_per, m_per), :] = jnp.dot(
                chunk, w_ref[...], preferred_element_type=jnp.float32)

        for h in range(R_HOPS):
            s = h % 2
            r = (h + 1) % 2
            l_active = h < L_HOPS
            if h >= 1:
                pl.semaphore_wait(r_credit, 1)
                if l_active:
                    pl.semaphore_wait(l_credit, 1)
            r_rdma = pltpu.make_async_remote_copy(
                src_ref=x_ref if h == 0 else rcomm.at[s],
                dst_ref=rcomm.at[r],
                send_sem=r_send_sems.at[s],
                recv_sem=r_recv_sems.at[r],
                device_id=(rgt,),
                device_id_type=pl.DeviceIdType.MESH,
            )
            r_rdma.start()
            if l_active:
                l_rdma = pltpu.make_async_remote_copy(
                    src_ref=x_ref if h == 0 else lcomm.at[s],
                    dst_ref=lcomm.at[r],
                    send_sem=l_send_sems.at[s],
                    recv_sem=l_recv_sems.at[r],
                    device_id=(lft,),
                    device_id_type=pl.DeviceIdType.MESH,
                )
                l_rdma.start()

            if h == 0:
                gemm_to(my_id, x_ref[...])
            else:
                gemm_to(prm[2 + (h - 1)], rcomm[s])
                gemm_to(prm[2 + R_HOPS + (h - 1)], lcomm[s])

            r_rdma.wait()
            if l_active:
                l_rdma.wait()

            if h < R_HOPS - 1:
                pl.semaphore_signal(
                    r_credit, inc=1,
                    device_id=(lft,), device_id_type=pl.DeviceIdType.MESH,
                )
            if l_active and h < L_HOPS - 1:
                pl.semaphore_signal(
                    l_credit, inc=1,
                    device_id=(rgt,), device_id_type=pl.DeviceIdType.MESH,
                )

        gemm_to(prm[2 + R_HOPS - 1], rcomm[R_HOPS % 2])

    out_shape = jax.ShapeDtypeStruct((N_DEV * m_per, n_per), jnp.float32)
    return pl.pallas_call(
        body,
        out_shape=out_shape,
        in_specs=[
            pl.BlockSpec(memory_space=pltpu.VMEM),
            pl.BlockSpec(memory_space=pltpu.VMEM),
            pl.BlockSpec(memory_space=pltpu.SMEM),
        ],
        out_specs=pl.BlockSpec(memory_space=pltpu.VMEM),
        scratch_shapes=[
            pltpu.VMEM((2, m_per, k), jnp.float32),
            pltpu.VMEM((2, m_per, k), jnp.float32),
            pltpu.SemaphoreType.DMA((2,)),
            pltpu.SemaphoreType.DMA((2,)),
            pltpu.SemaphoreType.DMA((2,)),
            pltpu.SemaphoreType.DMA((2,)),
            pltpu.SemaphoreType.REGULAR,
            pltpu.SemaphoreType.REGULAR,
        ],
        compiler_params=pltpu.CompilerParams(collective_id=0),
    )(x, w_mat, params)
